# baseline (speedup 1.0000x reference)
import jax
import jax.numpy as jnp
from jax import lax
from jax.experimental import pallas as pl
from jax.experimental.pallas import tpu as pltpu
from jax.experimental.pallas import tpu_sc as plsc

_VOCAB = 100000
_EMB = 64
_B = 4096
_L = 200

_NC = 2
_NS = 16
_NW = _NC * _NS
_LT = _L // 8
_BTS = _B // 128
_BG = _B // 512
_NBLK = _L * _BG
_BPW = _NBLK // _NW
_CH = 512
_PITCH = 521


def kernel(table, input_tensor):
    idx_t = input_tensor.T
    mesh = plsc.VectorSubcoreMesh(core_axis_name="c", subcore_axis_name="s")

    @pl.kernel(
        out_type=jax.ShapeDtypeStruct((_EMB, _LT, _BTS, 8, 128), jnp.float32),
        mesh=mesh,
        compiler_params=pltpu.CompilerParams(use_tc_tiling_on_sc=False,
                                             needs_layout_passes=False),
        scratch_types=[
            pltpu.VMEM((_CH,), jnp.int32),
            pltpu.VMEM((_CH,), jnp.int32),
            pltpu.VMEM((_CH, _EMB), jnp.float32),
            pltpu.VMEM((_CH, _EMB), jnp.float32),
            pltpu.VMEM((_EMB, _PITCH), jnp.float32),
            pltpu.SemaphoreType.DMA,
            pltpu.SemaphoreType.DMA,
            pltpu.SemaphoreType.DMA,
        ],
    )
    def sc_kernel(table_hbm, idx_hbm, out_hbm,
                  idx0, idx1, rows0, rows1, outT_v, sem0, sem1, sem2):
        wid = lax.axis_index("s") * _NC + lax.axis_index("c")
        g0 = wid * _BPW
        iota16 = lax.iota(jnp.int32, 16)
        row_ids = [iota16 + h0 * 16 for h0 in range(_EMB // 16)]

        def out_slices(g):
            l = g // _BG
            bg = g % _BG
            return [
                (outT_v.at[:, pl.ds(btp * 128, 128)],
                 out_hbm.at[:, l // 8, bg * 4 + btp, l % 8, :])
                for btp in range(4)
            ]

        def issue(g, idx_v, rows_v, sem):
            l = g // _BG
            bg = g % _BG
            pltpu.sync_copy(idx_hbm.at[l, pl.ds(bg * _CH, _CH)], idx_v)
            pltpu.async_copy(table_hbm.at[idx_v], rows_v, sem)

        def gwait(idx_v, rows_v, sem):
            pltpu.make_async_copy(table_hbm.at[idx_v], rows_v, sem).wait()

        def scat(rows_v):
            @plsc.parallel_loop(0, _CH, unroll=16)
            def _(r):
                col = jnp.full((16,), r, jnp.int32)
                for h0 in range(_EMB // 16):
                    x = rows_v[r, pl.ds(h0 * 16, 16)]
                    plsc.store_scatter(outT_v, [row_ids[h0], col], x)

        def out_issue(g):
            for src, dst in out_slices(g):
                pltpu.async_copy(src, dst, sem2)

        def out_drain(g):
            for src, dst in out_slices(g):
                pltpu.make_async_copy(src, dst, sem2).wait()

        issue(g0 + 0, idx0, rows0, sem0)
        issue(g0 + 1, idx1, rows1, sem1)

        gwait(idx0, rows0, sem0)
        scat(rows0)
        issue(g0 + 2, idx0, rows0, sem0)
        out_issue(g0 + 0)
        gwait(idx1, rows1, sem1)
        out_drain(g0 + 0)
        scat(rows1)
        issue(g0 + 3, idx1, rows1, sem1)
        out_issue(g0 + 1)

        @pl.loop(1, _BPW // 2 - 1)
        def _(i):
            e = g0 + 2 * i
            gwait(idx0, rows0, sem0)
            out_drain(e - 1)
            scat(rows0)
            issue(e + 2, idx0, rows0, sem0)
            out_issue(e)
            gwait(idx1, rows1, sem1)
            out_drain(e)
            scat(rows1)
            issue(e + 3, idx1, rows1, sem1)
            out_issue(e + 1)

        e_last = g0 + _BPW - 2
        gwait(idx0, rows0, sem0)
        out_drain(e_last - 1)
        scat(rows0)
        out_issue(e_last)
        gwait(idx1, rows1, sem1)
        out_drain(e_last)
        scat(rows1)
        out_issue(e_last + 1)
        out_drain(e_last + 1)

    out5 = sc_kernel(table, idx_t)
    return out5.transpose(2, 4, 0, 1, 3).reshape(_B, _EMB, _L)


# speedup vs baseline: 1.0012x; 1.0012x over previous
import jax
import jax.numpy as jnp
from jax import lax
from jax.experimental import pallas as pl
from jax.experimental.pallas import tpu as pltpu
from jax.experimental.pallas import tpu_sc as plsc

_VOCAB = 100000
_EMB = 64
_B = 4096
_L = 200

_NC = 2
_NS = 16
_NW = _NC * _NS
_LT = _L // 8
_BTS = _B // 128
_BG = _B // 512
_NBLK = _L * _BG
_BPW = _NBLK // _NW
_CH = 512
_PITCH = 521


def kernel(table, input_tensor):
    idx_t = input_tensor.reshape(_BTS, 128, _LT, 8).transpose(2, 0, 3, 1)
    mesh = plsc.VectorSubcoreMesh(core_axis_name="c", subcore_axis_name="s")

    @pl.kernel(
        out_type=jax.ShapeDtypeStruct((_EMB, _LT, _BTS, 8, 128), jnp.float32),
        mesh=mesh,
        compiler_params=pltpu.CompilerParams(use_tc_tiling_on_sc=False,
                                             needs_layout_passes=False),
        scratch_types=[
            pltpu.VMEM((4, 128), jnp.int32),
            pltpu.VMEM((4, 128), jnp.int32),
            pltpu.VMEM((_CH, _EMB), jnp.float32),
            pltpu.VMEM((_CH, _EMB), jnp.float32),
            pltpu.VMEM((_EMB, _PITCH), jnp.float32),
            pltpu.SemaphoreType.DMA,
            pltpu.SemaphoreType.DMA,
            pltpu.SemaphoreType.DMA,
        ],
    )
    def sc_kernel(table_hbm, idx_hbm, out_hbm,
                  idx0, idx1, rows0, rows1, outT_v, sem0, sem1, sem2):
        wid = lax.axis_index("s") * _NC + lax.axis_index("c")
        g0 = wid * _BPW
        iota16 = lax.iota(jnp.int32, 16)
        row_ids = [iota16 + h0 * 16 for h0 in range(_EMB // 16)]

        def out_slices(g):
            l = g // _BG
            bg = g % _BG
            return [
                (outT_v.at[:, pl.ds(btp * 128, 128)],
                 out_hbm.at[:, l // 8, bg * 4 + btp, l % 8, :])
                for btp in range(4)
            ]

        def issue(g, idx_v, rows_v, sem):
            l = g // _BG
            bg = g % _BG
            pltpu.sync_copy(
                idx_hbm.at[l // 8, pl.ds(bg * 4, 4), l % 8, :], idx_v)
            for j in range(4):
                pltpu.async_copy(table_hbm.at[idx_v.at[j]],
                                 rows_v.at[pl.ds(j * 128, 128), :], sem)

        def gwait(idx_v, rows_v, sem):
            for j in range(4):
                pltpu.make_async_copy(table_hbm.at[idx_v.at[j]],
                                      rows_v.at[pl.ds(j * 128, 128), :],
                                      sem).wait()

        def scat(rows_v):
            @plsc.parallel_loop(0, _CH, unroll=16)
            def _(r):
                col = jnp.full((16,), r, jnp.int32)
                for h0 in range(_EMB // 16):
                    x = rows_v[r, pl.ds(h0 * 16, 16)]
                    plsc.store_scatter(outT_v, [row_ids[h0], col], x)

        def out_issue(g):
            for src, dst in out_slices(g):
                pltpu.async_copy(src, dst, sem2)

        def out_drain(g):
            for src, dst in out_slices(g):
                pltpu.make_async_copy(src, dst, sem2).wait()

        issue(g0 + 0, idx0, rows0, sem0)
        issue(g0 + 1, idx1, rows1, sem1)

        gwait(idx0, rows0, sem0)
        scat(rows0)
        issue(g0 + 2, idx0, rows0, sem0)
        out_issue(g0 + 0)
        gwait(idx1, rows1, sem1)
        out_drain(g0 + 0)
        scat(rows1)
        issue(g0 + 3, idx1, rows1, sem1)
        out_issue(g0 + 1)

        @pl.loop(1, _BPW // 2 - 1)
        def _(i):
            e = g0 + 2 * i
            gwait(idx0, rows0, sem0)
            out_drain(e - 1)
            scat(rows0)
            issue(e + 2, idx0, rows0, sem0)
            out_issue(e)
            gwait(idx1, rows1, sem1)
            out_drain(e)
            scat(rows1)
            issue(e + 3, idx1, rows1, sem1)
            out_issue(e + 1)

        e_last = g0 + _BPW - 2
        gwait(idx0, rows0, sem0)
        out_drain(e_last - 1)
        scat(rows0)
        out_issue(e_last)
        gwait(idx1, rows1, sem1)
        out_drain(e_last)
        scat(rows1)
        out_issue(e_last + 1)
        out_drain(e_last + 1)

    out5 = sc_kernel(table, idx_t)
    return out5.transpose(2, 4, 0, 1, 3).reshape(_B, _EMB, _L)


# speedup vs baseline: 1.1714x; 1.1699x over previous
import jax
import jax.numpy as jnp
from jax import lax
from jax.experimental import pallas as pl
from jax.experimental.pallas import tpu as pltpu
from jax.experimental.pallas import tpu_sc as plsc

_VOCAB = 100000
_EMB = 64
_B = 4096
_L = 200

_NC = 2
_NS = 16
_NW = _NC * _NS
_LT = _L // 8
_BTS = _B // 128
_BG = _B // 512
_NBLK = _L * _BG
_BPW = _NBLK // _NW
_CH = 512
_PITCH = 521


def kernel(table, input_tensor):
    idx_t = input_tensor.reshape(_BTS, 128, _LT, 8).transpose(2, 0, 3, 1)
    mesh = plsc.VectorSubcoreMesh(core_axis_name="c", subcore_axis_name="s")

    @pl.kernel(
        out_type=jax.ShapeDtypeStruct((_EMB, _LT, _BTS, 8, 128), jnp.float32),
        mesh=mesh,
        compiler_params=pltpu.CompilerParams(use_tc_tiling_on_sc=False,
                                             needs_layout_passes=False),
        scratch_types=[
            pltpu.VMEM((4, 128), jnp.int32),
            pltpu.VMEM((4, 128), jnp.int32),
            pltpu.VMEM((_CH, _EMB), jnp.float32),
            pltpu.VMEM((_CH, _EMB), jnp.float32),
            pltpu.VMEM((_EMB, _PITCH), jnp.float32),
            pltpu.SemaphoreType.DMA,
            pltpu.SemaphoreType.DMA,
            pltpu.SemaphoreType.DMA,
            pltpu.SemaphoreType.DMA,
            pltpu.SemaphoreType.DMA,
        ],
    )
    def sc_kernel(table_hbm, idx_hbm, out_hbm,
                  idx0, idx1, rows0, rows1, outT_v,
                  sem0, sem1, sem2, semi0, semi1):
        wid = lax.axis_index("s") * _NC + lax.axis_index("c")
        g0 = wid * _BPW
        iota16 = lax.iota(jnp.int32, 16)
        row_ids = [iota16 + h0 * 16 for h0 in range(_EMB // 16)]

        def idx_src(g):
            l = g // _BG
            bg = g % _BG
            return idx_hbm.at[l // 8, pl.ds(bg * 4, 4), l % 8, :]

        def idx_fetch(g, idx_v, semi):
            pltpu.async_copy(idx_src(g), idx_v, semi)

        def idx_wait(g, idx_v, semi):
            pltpu.make_async_copy(idx_src(g), idx_v, semi).wait()

        def gather_issue(idx_v, rows_v, sem):
            for j in range(4):
                pltpu.async_copy(table_hbm.at[idx_v.at[j]],
                                 rows_v.at[pl.ds(j * 128, 128), :], sem)

        def gwait(idx_v, rows_v, sem):
            for j in range(4):
                pltpu.make_async_copy(table_hbm.at[idx_v.at[j]],
                                      rows_v.at[pl.ds(j * 128, 128), :],
                                      sem).wait()

        def scat(rows_v):
            @plsc.parallel_loop(0, _CH, unroll=16)
            def _(r):
                col = jnp.full((16,), r, jnp.int32)
                for h0 in range(_EMB // 16):
                    x = rows_v[r, pl.ds(h0 * 16, 16)]
                    plsc.store_scatter(outT_v, [row_ids[h0], col], x)

        def out_slices(g):
            l = g // _BG
            bg = g % _BG
            return [
                (outT_v.at[:, pl.ds(btp * 128, 128)],
                 out_hbm.at[:, l // 8, bg * 4 + btp, l % 8, :])
                for btp in range(4)
            ]

        def out_issue(g):
            for src, dst in out_slices(g):
                pltpu.async_copy(src, dst, sem2)

        def out_drain(g):
            for src, dst in out_slices(g):
                pltpu.make_async_copy(src, dst, sem2).wait()

        idx_fetch(g0 + 0, idx0, semi0)
        idx_fetch(g0 + 1, idx1, semi1)
        idx_wait(g0 + 0, idx0, semi0)
        gather_issue(idx0, rows0, sem0)
        idx_wait(g0 + 1, idx1, semi1)
        gather_issue(idx1, rows1, sem1)

        gwait(idx0, rows0, sem0)
        idx_fetch(g0 + 2, idx0, semi0)
        scat(rows0)
        idx_wait(g0 + 2, idx0, semi0)
        gather_issue(idx0, rows0, sem0)
        out_issue(g0 + 0)
        gwait(idx1, rows1, sem1)
        idx_fetch(g0 + 3, idx1, semi1)
        out_drain(g0 + 0)
        scat(rows1)
        idx_wait(g0 + 3, idx1, semi1)
        gather_issue(idx1, rows1, sem1)
        out_issue(g0 + 1)

        @pl.loop(1, _BPW // 2 - 1)
        def _(i):
            e = g0 + 2 * i
            gwait(idx0, rows0, sem0)
            idx_fetch(e + 2, idx0, semi0)
            out_drain(e - 1)
            scat(rows0)
            idx_wait(e + 2, idx0, semi0)
            gather_issue(idx0, rows0, sem0)
            out_issue(e)
            gwait(idx1, rows1, sem1)
            idx_fetch(e + 3, idx1, semi1)
            out_drain(e)
            scat(rows1)
            idx_wait(e + 3, idx1, semi1)
            gather_issue(idx1, rows1, sem1)
            out_issue(e + 1)

        e_last = g0 + _BPW - 2
        gwait(idx0, rows0, sem0)
        out_drain(e_last - 1)
        scat(rows0)
        out_issue(e_last)
        gwait(idx1, rows1, sem1)
        out_drain(e_last)
        scat(rows1)
        out_issue(e_last + 1)
        out_drain(e_last + 1)

    out5 = sc_kernel(table, idx_t)
    return out5.transpose(2, 4, 0, 1, 3).reshape(_B, _EMB, _L)
